# Initial kernel scaffold; baseline (speedup 1.0000x reference)
import jax
import jax.numpy as jnp
from jax.experimental import pallas as pl


def kernel(z, t, conditioning, mask, W_c1, b_c1, W_c2, b_c2, W_c3, b_c3, W_embed, b_embed, W_mp0, b_mp0, W_mp, b_mp, W_dec, b_dec):
    raise NotImplementedError("write your pallas kernel here")


# speedup vs baseline: 23.4519x; 23.4519x over previous
import functools

import jax
import jax.numpy as jnp
from jax import lax
from jax.experimental import pallas as pl
from jax.experimental.pallas import tpu as pltpu
from jax.experimental.pallas import tpu_sc as plsc

N = 10000
NP = 10240
K = 20
L = 128
BQ = 512
BM = 1024
CH = 128
NCHUNK = NP // CH
NC, NS = 2, 16
WROWS = NP // NS
CPC = NCHUNK // NC
BIGF = 3.0e38


def _knn_body(q_ref, pT_ref, out_ref):
    q = q_ref[...]
    pT = pT_ref[...]
    acc = jnp.zeros((BQ, NP), jnp.float32)
    for c in range(3):
        d = q[:, c:c + 1] - pT[c:c + 1, :]
        acc = acc + d * d
    colid = lax.broadcasted_iota(jnp.int32, (BQ, NP), 1)
    acc = jnp.where(colid >= N, BIGF, acc)
    lane = lax.broadcasted_iota(jnp.int32, (BQ, 128), 1)

    def body(k, carry):
        a, ob = carry
        m = jnp.min(a, axis=1, keepdims=True)
        sel = jnp.where(a == m, colid, jnp.int32(NP))
        idxk = jnp.min(sel, axis=1, keepdims=True)
        ob = jnp.where(lane == k, idxk, ob)
        a = jnp.where(colid == idxk, BIGF, a)
        return a, ob

    _, outbuf = lax.fori_loop(
        0, K, body, (acc, jnp.zeros((BQ, 128), jnp.int32)))
    out_ref[...] = outbuf[:, :K]


def _knn(qpad, pT):
    return pl.pallas_call(
        _knn_body,
        grid=(NP // BQ,),
        in_specs=[
            pl.BlockSpec((BQ, 8), lambda i: (i, 0)),
            pl.BlockSpec((8, NP), lambda i: (0, 0)),
        ],
        out_specs=pl.BlockSpec((BQ, K), lambda i: (i, 0)),
        out_shape=jax.ShapeDtypeStruct((NP, K), jnp.int32),
    )(qpad, pT)


def _cond_body(t_ref, freq_ref, c_ref, w1a_ref, w1b_ref, w1c_ref, b1_ref,
               w2_ref, b2_ref, w3_ref, b3_ref, we2_ref, be_ref, out_ref):
    targ = t_ref[...] * freq_ref[...]
    f32 = jnp.float32
    x = (jnp.dot(jnp.sin(targ), w1a_ref[...], preferred_element_type=f32)
         + jnp.dot(jnp.cos(targ), w1b_ref[...], preferred_element_type=f32)
         + jnp.dot(c_ref[...], w1c_ref[...], preferred_element_type=f32)
         + b1_ref[...])
    x = jax.nn.gelu(x)
    x = jax.nn.gelu(jnp.dot(x, w2_ref[...], preferred_element_type=f32)
                    + b2_ref[...])
    x = jnp.dot(x, w3_ref[...], preferred_element_type=f32) + b3_ref[...]
    out_ref[...] = (jnp.dot(x, we2_ref[...], preferred_element_type=f32)
                    + be_ref[...])


def _cond(t, freq, conditioning, w1a, w1b, w1c, b1, w2, b2, w3, b3, we2, be):
    return pl.pallas_call(
        _cond_body,
        out_shape=jax.ShapeDtypeStruct((1, L), jnp.float32),
    )(t, freq, conditioning, w1a, w1b, w1c, b1, w2, b2, w3, b3, we2, be)


def _embed_body(z_ref, we1_ref, bp_ref, out_ref):
    out_ref[...] = jax.nn.gelu(
        jnp.dot(z_ref[...], we1_ref[...], preferred_element_type=jnp.float32)
        + bp_ref[...])


def _embed(zp8, we1, bp):
    return pl.pallas_call(
        _embed_body,
        grid=(NP // BM,),
        in_specs=[
            pl.BlockSpec((BM, 8), lambda i: (i, 0)),
            pl.BlockSpec((8, L), lambda i: (0, 0)),
            pl.BlockSpec((1, L), lambda i: (0, 0)),
        ],
        out_specs=pl.BlockSpec((BM, L), lambda i: (i, 0)),
        out_shape=jax.ShapeDtypeStruct((NP, L), jnp.float32),
    )(zp8, we1, bp)


def _agg_body(h_hbm, idx_hbm, zro_hbm, out_hbm, hbuf, ibuf, aggsh):
    core = lax.axis_index("c")
    sid = lax.axis_index("s")
    pltpu.sync_copy(zro_hbm, aggsh.at[pl.ds(sid * WROWS, WROWS)])
    plsc.subcore_barrier()
    for j in range(3):
        cl = j * NS + sid

        @pl.when(cl < CPC)
        def _():
            cid = core * CPC + cl
            pltpu.sync_copy(h_hbm.at[pl.ds(cid * CH, CH)], hbuf)
            pltpu.sync_copy(idx_hbm.at[cid], ibuf)
            for k in range(K):
                pltpu.sync_copy(hbuf, aggsh.at[ibuf.at[k]], add=True)

    plsc.subcore_barrier()
    pltpu.sync_copy(aggsh.at[pl.ds(sid * WROWS, WROWS)],
                    out_hbm.at[core, pl.ds(sid * WROWS, WROWS)])


@functools.cache
def _agg_kernel():
    return functools.partial(
        pl.kernel,
        mesh=plsc.VectorSubcoreMesh(core_axis_name="c", subcore_axis_name="s"),
        out_type=jax.ShapeDtypeStruct((NC, NP, L), jnp.float32),
        scratch_types=[
            pltpu.VMEM((CH, L), jnp.float32),
            pltpu.VMEM((K, CH), jnp.int32),
            pltpu.VMEM_SHARED((NP, L), jnp.float32),
        ],
    )(_agg_body)


def _agg(h, idx3, zeros):
    return _agg_kernel()(h, idx3, zeros)


def _step_body_last(h_ref, p0_ref, p1_ref, z_ref, w0h_ref, w0a_ref, b0_ref,
                    w1_ref, b1_ref, w2_ref, b2_ref, w3_ref, b3_ref,
                    wd_ref, bd_ref, out_ref):
    _step_common(True, h_ref, p0_ref, p1_ref, z_ref, w0h_ref, w0a_ref, b0_ref,
                 w1_ref, b1_ref, w2_ref, b2_ref, w3_ref, b3_ref,
                 wd_ref, bd_ref, out_ref)


def _step_body_mid(h_ref, p0_ref, p1_ref, z_ref, w0h_ref, w0a_ref, b0_ref,
                   w1_ref, b1_ref, w2_ref, b2_ref, w3_ref, b3_ref,
                   wd_ref, bd_ref, out_ref):
    _step_common(False, h_ref, p0_ref, p1_ref, z_ref, w0h_ref, w0a_ref, b0_ref,
                 w1_ref, b1_ref, w2_ref, b2_ref, w3_ref, b3_ref,
                 wd_ref, bd_ref, out_ref)


def _step_common(last, h_ref, p0_ref, p1_ref, z_ref, w0h_ref, w0a_ref, b0_ref,
                 w1_ref, b1_ref, w2_ref, b2_ref, w3_ref, b3_ref,
                 wd_ref, bd_ref, out_ref):
    f32 = jnp.float32
    h = h_ref[...]
    agg = p0_ref[...] + p1_ref[...]
    u = jax.nn.gelu(jnp.dot(h, w0h_ref[...], preferred_element_type=f32)
                    + jnp.dot(agg, w0a_ref[...], preferred_element_type=f32)
                    + b0_ref[...])
    u = jax.nn.gelu(jnp.dot(u, w1_ref[...], preferred_element_type=f32)
                    + b1_ref[...])
    u = jax.nn.gelu(jnp.dot(u, w2_ref[...], preferred_element_type=f32)
                    + b2_ref[...])
    u = jax.nn.gelu(jnp.dot(u, w3_ref[...], preferred_element_type=f32)
                    + b3_ref[...])
    hn = h + u
    if last:
        out_ref[...] = (z_ref[...]
                        + jnp.dot(hn, wd_ref[...], preferred_element_type=f32)
                        + bd_ref[...])
    else:
        out_ref[...] = hn


def _step(h, p0, p1, zp8, w0h, w0a, b0, w1, b1, w2, b2, w3, b3, wd, bd, last):
    out_lanes = 8 if last else L
    body = _step_body_last if last else _step_body_mid
    return pl.pallas_call(
        body,
        grid=(NP // BM,),
        in_specs=[
            pl.BlockSpec((BM, L), lambda i: (i, 0)),
            pl.BlockSpec((BM, L), lambda i: (i, 0)),
            pl.BlockSpec((BM, L), lambda i: (i, 0)),
            pl.BlockSpec((BM, 8), lambda i: (i, 0)),
            pl.BlockSpec((L, L), lambda i: (0, 0)),
            pl.BlockSpec((L, L), lambda i: (0, 0)),
            pl.BlockSpec((1, L), lambda i: (0, 0)),
            pl.BlockSpec((L, L), lambda i: (0, 0)),
            pl.BlockSpec((1, L), lambda i: (0, 0)),
            pl.BlockSpec((L, L), lambda i: (0, 0)),
            pl.BlockSpec((1, L), lambda i: (0, 0)),
            pl.BlockSpec((L, L), lambda i: (0, 0)),
            pl.BlockSpec((1, L), lambda i: (0, 0)),
            pl.BlockSpec((L, 8), lambda i: (0, 0)),
            pl.BlockSpec((1, 8), lambda i: (0, 0)),
        ],
        out_specs=pl.BlockSpec((BM, out_lanes), lambda i: (i, 0)),
        out_shape=jax.ShapeDtypeStruct((NP, out_lanes), jnp.float32),
    )(h, p0, p1, zp8, w0h, w0a, b0, w1, b1, w2, b2, w3, b3, wd, bd)


def kernel(z, t, conditioning, mask, W_c1, b_c1, W_c2, b_c2, W_c3, b_c3,
           W_embed, b_embed, W_mp0, b_mp0, W_mp, b_mp, W_dec, b_dec):
    z0 = z[0]
    zp8 = jnp.pad(z0, ((0, NP - N), (0, 1)))
    qpad = jnp.pad(z0[:, :3], ((0, NP - N), (0, 5)))
    pT = qpad.T

    idx = _knn(qpad, pT)
    rows = jnp.arange(NP, dtype=jnp.int32)[:, None]
    idxf = jnp.where(rows < N, idx, NP - 1)
    idx3 = idxf.reshape(NCHUNK, CH, K).transpose(0, 2, 1)

    half = 16
    freq = jnp.exp(-jnp.log(10000.0)
                   * jnp.arange(half, dtype=jnp.float32) / (half - 1))
    bp = _cond(
        t.reshape(1, 1), freq.reshape(1, half), conditioning,
        W_c1[:half], W_c1[half:2 * half], W_c1[2 * half:], b_c1.reshape(1, -1),
        W_c2, b_c2.reshape(1, -1), W_c3, b_c3.reshape(1, -1),
        W_embed[7:], b_embed.reshape(1, L))

    we1 = jnp.pad(W_embed[:7], ((0, 1), (0, 0)))
    h = _embed(zp8, we1, bp)

    zeros = jnp.zeros((WROWS, L), jnp.float32)
    wd = jnp.pad(W_dec, ((0, 0), (0, 1)))
    bd = jnp.pad(b_dec, (0, 1)).reshape(1, 8)

    n_steps = W_mp0.shape[0]
    out = None
    for s in range(n_steps):
        parts = _agg(h, idx3, zeros)
        res = _step(h, parts[0], parts[1], zp8,
                    W_mp0[s, :L], W_mp0[s, L:], b_mp0[s].reshape(1, L),
                    W_mp[s, 0], b_mp[s, 0].reshape(1, L),
                    W_mp[s, 1], b_mp[s, 1].reshape(1, L),
                    W_mp[s, 2], b_mp[s, 2].reshape(1, L),
                    wd, bd, last=(s == n_steps - 1))
        if s == n_steps - 1:
            out = res
        else:
            h = res

    return out[:N, :7][None]


# speedup vs baseline: 23.4831x; 1.0013x over previous
import functools

import jax
import jax.numpy as jnp
from jax import lax
from jax.experimental import pallas as pl
from jax.experimental.pallas import tpu as pltpu
from jax.experimental.pallas import tpu_sc as plsc

N = 10000
NP = 10240
K = 20
L = 128
BQ = 512
BM = 1024
CH = 128
NCHUNK = NP // CH
NC, NS = 2, 16
WROWS = NP // NS
CPC = NCHUNK // NC
BIGF = 3.0e38


def _knn_body(q_ref, pT_ref, out_ref):
    q = q_ref[...]
    pT = pT_ref[...]
    acc = jnp.zeros((BQ, NP), jnp.float32)
    for c in range(3):
        d = q[:, c:c + 1] - pT[c:c + 1, :]
        acc = acc + d * d
    colid = lax.broadcasted_iota(jnp.int32, (BQ, NP), 1)
    acc = jnp.where(colid >= N, BIGF, acc)
    lane = lax.broadcasted_iota(jnp.int32, (BQ, 128), 1)

    def body(k, carry):
        a, ob = carry
        m = jnp.min(a, axis=1, keepdims=True)
        sel = jnp.where(a == m, colid, jnp.int32(NP))
        idxk = jnp.min(sel, axis=1, keepdims=True)
        ob = jnp.where(lane == k, idxk, ob)
        a = jnp.where(colid == idxk, BIGF, a)
        return a, ob

    _, outbuf = lax.fori_loop(
        0, K, body, (acc, jnp.zeros((BQ, 128), jnp.int32)))
    out_ref[...] = outbuf[:, :K]


def _knn(qpad, pT):
    return pl.pallas_call(
        _knn_body,
        grid=(NP // BQ,),
        in_specs=[
            pl.BlockSpec((BQ, 8), lambda i: (i, 0)),
            pl.BlockSpec((8, NP), lambda i: (0, 0)),
        ],
        out_specs=pl.BlockSpec((BQ, K), lambda i: (i, 0)),
        out_shape=jax.ShapeDtypeStruct((NP, K), jnp.int32),
    )(qpad, pT)


def _cond_body(t_ref, freq_ref, c_ref, w1a_ref, w1b_ref, w1c_ref, b1_ref,
               w2_ref, b2_ref, w3_ref, b3_ref, we2_ref, be_ref, out_ref):
    targ = t_ref[...] * freq_ref[...]
    f32 = jnp.float32
    x = (jnp.dot(jnp.sin(targ), w1a_ref[...], preferred_element_type=f32)
         + jnp.dot(jnp.cos(targ), w1b_ref[...], preferred_element_type=f32)
         + jnp.dot(c_ref[...], w1c_ref[...], preferred_element_type=f32)
         + b1_ref[...])
    x = jax.nn.gelu(x)
    x = jax.nn.gelu(jnp.dot(x, w2_ref[...], preferred_element_type=f32)
                    + b2_ref[...])
    x = jnp.dot(x, w3_ref[...], preferred_element_type=f32) + b3_ref[...]
    out_ref[...] = (jnp.dot(x, we2_ref[...], preferred_element_type=f32)
                    + be_ref[...])


def _cond(t, freq, conditioning, w1a, w1b, w1c, b1, w2, b2, w3, b3, we2, be):
    return pl.pallas_call(
        _cond_body,
        out_shape=jax.ShapeDtypeStruct((1, L), jnp.float32),
    )(t, freq, conditioning, w1a, w1b, w1c, b1, w2, b2, w3, b3, we2, be)


def _embed_body(z_ref, we1_ref, bp_ref, out_ref):
    out_ref[...] = jax.nn.gelu(
        jnp.dot(z_ref[...], we1_ref[...], preferred_element_type=jnp.float32)
        + bp_ref[...])


def _embed(zp8, we1, bp):
    return pl.pallas_call(
        _embed_body,
        grid=(NP // BM,),
        in_specs=[
            pl.BlockSpec((BM, 8), lambda i: (i, 0)),
            pl.BlockSpec((8, L), lambda i: (0, 0)),
            pl.BlockSpec((1, L), lambda i: (0, 0)),
        ],
        out_specs=pl.BlockSpec((BM, L), lambda i: (i, 0)),
        out_shape=jax.ShapeDtypeStruct((NP, L), jnp.float32),
    )(zp8, we1, bp)


def _agg_body(h_hbm, idx_hbm, zro_hbm, out_hbm, hbuf, ibuf, aggsh, sem):
    core = lax.axis_index("c")
    sid = lax.axis_index("s")
    pltpu.sync_copy(zro_hbm, aggsh.at[pl.ds(sid * WROWS, WROWS)])
    plsc.subcore_barrier()
    for j in range(3):
        cl = j * NS + sid

        @pl.when(cl < CPC)
        def _():
            cid = core * CPC + cl
            pltpu.sync_copy(h_hbm.at[pl.ds(cid * CH, CH)], hbuf)
            pltpu.sync_copy(idx_hbm.at[cid], ibuf)
            cps = [pltpu.async_copy(hbuf, aggsh.at[ibuf.at[k]], sem,
                                    add=True) for k in range(K)]
            for cp in cps:
                cp.wait()

    plsc.subcore_barrier()
    pltpu.sync_copy(aggsh.at[pl.ds(sid * WROWS, WROWS)],
                    out_hbm.at[core, pl.ds(sid * WROWS, WROWS)])


@functools.cache
def _agg_kernel():
    return functools.partial(
        pl.kernel,
        mesh=plsc.VectorSubcoreMesh(core_axis_name="c", subcore_axis_name="s"),
        out_type=jax.ShapeDtypeStruct((NC, NP, L), jnp.float32),
        scratch_types=[
            pltpu.VMEM((CH, L), jnp.float32),
            pltpu.VMEM((K, CH), jnp.int32),
            pltpu.VMEM_SHARED((NP, L), jnp.float32),
            pltpu.SemaphoreType.DMA,
        ],
    )(_agg_body)


def _agg(h, idx3, zeros):
    return _agg_kernel()(h, idx3, zeros)


def _step_body_last(h_ref, p0_ref, p1_ref, z_ref, w0h_ref, w0a_ref, b0_ref,
                    w1_ref, b1_ref, w2_ref, b2_ref, w3_ref, b3_ref,
                    wd_ref, bd_ref, out_ref):
    _step_common(True, h_ref, p0_ref, p1_ref, z_ref, w0h_ref, w0a_ref, b0_ref,
                 w1_ref, b1_ref, w2_ref, b2_ref, w3_ref, b3_ref,
                 wd_ref, bd_ref, out_ref)


def _step_body_mid(h_ref, p0_ref, p1_ref, z_ref, w0h_ref, w0a_ref, b0_ref,
                   w1_ref, b1_ref, w2_ref, b2_ref, w3_ref, b3_ref,
                   wd_ref, bd_ref, out_ref):
    _step_common(False, h_ref, p0_ref, p1_ref, z_ref, w0h_ref, w0a_ref, b0_ref,
                 w1_ref, b1_ref, w2_ref, b2_ref, w3_ref, b3_ref,
                 wd_ref, bd_ref, out_ref)


def _step_common(last, h_ref, p0_ref, p1_ref, z_ref, w0h_ref, w0a_ref, b0_ref,
                 w1_ref, b1_ref, w2_ref, b2_ref, w3_ref, b3_ref,
                 wd_ref, bd_ref, out_ref):
    f32 = jnp.float32
    h = h_ref[...]
    agg = p0_ref[...] + p1_ref[...]
    u = jax.nn.gelu(jnp.dot(h, w0h_ref[...], preferred_element_type=f32)
                    + jnp.dot(agg, w0a_ref[...], preferred_element_type=f32)
                    + b0_ref[...])
    u = jax.nn.gelu(jnp.dot(u, w1_ref[...], preferred_element_type=f32)
                    + b1_ref[...])
    u = jax.nn.gelu(jnp.dot(u, w2_ref[...], preferred_element_type=f32)
                    + b2_ref[...])
    u = jax.nn.gelu(jnp.dot(u, w3_ref[...], preferred_element_type=f32)
                    + b3_ref[...])
    hn = h + u
    if last:
        out_ref[...] = (z_ref[...]
                        + jnp.dot(hn, wd_ref[...], preferred_element_type=f32)
                        + bd_ref[...])
    else:
        out_ref[...] = hn


def _step(h, p0, p1, zp8, w0h, w0a, b0, w1, b1, w2, b2, w3, b3, wd, bd, last):
    out_lanes = 8 if last else L
    body = _step_body_last if last else _step_body_mid
    return pl.pallas_call(
        body,
        grid=(NP // BM,),
        in_specs=[
            pl.BlockSpec((BM, L), lambda i: (i, 0)),
            pl.BlockSpec((BM, L), lambda i: (i, 0)),
            pl.BlockSpec((BM, L), lambda i: (i, 0)),
            pl.BlockSpec((BM, 8), lambda i: (i, 0)),
            pl.BlockSpec((L, L), lambda i: (0, 0)),
            pl.BlockSpec((L, L), lambda i: (0, 0)),
            pl.BlockSpec((1, L), lambda i: (0, 0)),
            pl.BlockSpec((L, L), lambda i: (0, 0)),
            pl.BlockSpec((1, L), lambda i: (0, 0)),
            pl.BlockSpec((L, L), lambda i: (0, 0)),
            pl.BlockSpec((1, L), lambda i: (0, 0)),
            pl.BlockSpec((L, L), lambda i: (0, 0)),
            pl.BlockSpec((1, L), lambda i: (0, 0)),
            pl.BlockSpec((L, 8), lambda i: (0, 0)),
            pl.BlockSpec((1, 8), lambda i: (0, 0)),
        ],
        out_specs=pl.BlockSpec((BM, out_lanes), lambda i: (i, 0)),
        out_shape=jax.ShapeDtypeStruct((NP, out_lanes), jnp.float32),
    )(h, p0, p1, zp8, w0h, w0a, b0, w1, b1, w2, b2, w3, b3, wd, bd)


def kernel(z, t, conditioning, mask, W_c1, b_c1, W_c2, b_c2, W_c3, b_c3,
           W_embed, b_embed, W_mp0, b_mp0, W_mp, b_mp, W_dec, b_dec):
    z0 = z[0]
    zp8 = jnp.pad(z0, ((0, NP - N), (0, 1)))
    qpad = jnp.pad(z0[:, :3], ((0, NP - N), (0, 5)))
    pT = qpad.T

    idx = _knn(qpad, pT)
    rows = jnp.arange(NP, dtype=jnp.int32)[:, None]
    idxf = jnp.where(rows < N, idx, NP - 1)
    idx3 = idxf.reshape(NCHUNK, CH, K).transpose(0, 2, 1)

    half = 16
    freq = jnp.exp(-jnp.log(10000.0)
                   * jnp.arange(half, dtype=jnp.float32) / (half - 1))
    bp = _cond(
        t.reshape(1, 1), freq.reshape(1, half), conditioning,
        W_c1[:half], W_c1[half:2 * half], W_c1[2 * half:], b_c1.reshape(1, -1),
        W_c2, b_c2.reshape(1, -1), W_c3, b_c3.reshape(1, -1),
        W_embed[7:], b_embed.reshape(1, L))

    we1 = jnp.pad(W_embed[:7], ((0, 1), (0, 0)))
    h = _embed(zp8, we1, bp)

    zeros = jnp.zeros((WROWS, L), jnp.float32)
    wd = jnp.pad(W_dec, ((0, 0), (0, 1)))
    bd = jnp.pad(b_dec, (0, 1)).reshape(1, 8)

    n_steps = W_mp0.shape[0]
    out = None
    for s in range(n_steps):
        parts = _agg(h, idx3, zeros)
        res = _step(h, parts[0], parts[1], zp8,
                    W_mp0[s, :L], W_mp0[s, L:], b_mp0[s].reshape(1, L),
                    W_mp[s, 0], b_mp[s, 0].reshape(1, L),
                    W_mp[s, 1], b_mp[s, 1].reshape(1, L),
                    W_mp[s, 2], b_mp[s, 2].reshape(1, L),
                    wd, bd, last=(s == n_steps - 1))
        if s == n_steps - 1:
            out = res
        else:
            h = res

    return out[:N, :7][None]
